# baseline (speedup 1.0000x reference)
import functools

import jax
import jax.numpy as jnp
from jax import lax
from jax.experimental import pallas as pl
from jax.experimental.pallas import tpu as pltpu
from jax.experimental.pallas import tpu_sc as plsc

_BINS = 64
_H = 512
_W = 512
_NPIX = float(_H * _W)
_CHUNK = 16384
_NCHUNK = (_H * _W) // _CHUNK


def _dense_stats(img):
    r = img[0]
    g = img[1]
    b = img[2]
    s0 = jnp.sum(r)
    s1 = jnp.sum(g)
    s2 = jnp.sum(b)
    gray = 0.299 * r + 0.587 * g + 0.114 * b
    mn = jnp.min(gray)
    mx = jnp.max(gray)
    denom = mx - mn
    safe = jnp.where(denom > 0, denom, 1.0)
    xn = (gray - mn) / safe
    bidx = (xn * (_BINS - 1)).astype(jnp.int32)
    bidx = jnp.clip(bidx, 0, _BINS - 1)
    return s0, s1, s2, mn, mx, bidx * 16


def _stage_a_kernel(x_ref, y_ref, ix_ref, iy_ref, stats_ref):
    xs0, xs1, xs2, xmn, xmx, xidx = _dense_stats(x_ref[0])
    ys0, ys1, ys2, ymn, ymx, yidx = _dense_stats(y_ref[0])

    ix_ref[...] = xidx[None]
    iy_ref[...] = yidx[None]

    col = jax.lax.broadcasted_iota(jnp.int32, (1, 1, 16), 2)
    row = jnp.zeros((1, 1, 16), jnp.float32)
    for k, v in enumerate((xs0, xs1, xs2, xmn, xmx, ys0, ys1, ys2, ymn, ymx)):
        row = jnp.where(col == k, v, row)
    stats_ref[...] = row


def _stage_a(x, y):
    B = x.shape[0]
    return pl.pallas_call(
        _stage_a_kernel,
        grid=(B,),
        in_specs=[
            pl.BlockSpec((1, 3, _H, _W), lambda i: (i, 0, 0, 0)),
            pl.BlockSpec((1, 3, _H, _W), lambda i: (i, 0, 0, 0)),
        ],
        out_specs=[
            pl.BlockSpec((1, _H, _W), lambda i: (i, 0, 0)),
            pl.BlockSpec((1, _H, _W), lambda i: (i, 0, 0)),
            pl.BlockSpec((1, 1, 16), lambda i: (i, 0, 0)),
        ],
        out_shape=[
            jax.ShapeDtypeStruct((B, _H, _W), jnp.int32),
            jax.ShapeDtypeStruct((B, _H, _W), jnp.int32),
            jax.ShapeDtypeStruct((B, 1, 16), jnp.float32),
        ],
    )(x, y)


def _sc_hist(ix, iy):
    mesh = plsc.VectorSubcoreMesh(core_axis_name="c", subcore_axis_name="s")

    @functools.partial(
        pl.kernel,
        out_type=jax.ShapeDtypeStruct((2, 16, _BINS * 16), jnp.float32),
        mesh=mesh,
        scratch_types=[
            pltpu.VMEM((_CHUNK,), jnp.int32),
            pltpu.VMEM((_CHUNK,), jnp.int32),
            pltpu.VMEM((_BINS * 16,), jnp.float32),
            pltpu.SemaphoreType.DMA,
            pltpu.SemaphoreType.DMA,
        ],
        compiler_params=pltpu.CompilerParams(needs_layout_passes=False),
    )
    def run(ix_hbm, iy_hbm, out_hbm, buf0_v, buf1_v, hist_v, sem0, sem1):
        c = lax.axis_index("c")
        s = lax.axis_index("s")
        zeros16 = jnp.zeros((16,), jnp.float32)
        ones16 = jnp.ones((16,), jnp.float32)
        lanes = lax.iota(jnp.int32, 16)

        def process(ghbm):
            for b in range(_BINS):
                hist_v[pl.ds(b * 16, 16)] = zeros16

            bufs = (buf0_v, buf1_v)
            sems = (sem0, sem1)
            handles = {0: pltpu.async_copy(ghbm.at[s, 0], bufs[0], sems[0])}
            for ch in range(_NCHUNK):
                nxt = ch + 1
                if nxt < _NCHUNK:
                    handles[nxt] = pltpu.async_copy(
                        ghbm.at[s, nxt], bufs[nxt % 2], sems[nxt % 2]
                    )
                handles[ch].wait()
                bufref = bufs[ch % 2]

                @plsc.parallel_loop(0, _CHUNK // 16, unroll=8)
                def _body(i, bufref=bufref):
                    bi16 = bufref[pl.ds(i * 16, 16)]
                    plsc.addupdate_scatter(hist_v, [bi16 + lanes], ones16)

            pltpu.sync_copy(hist_v, out_hbm.at[c, s])

        @pl.when(c == 0)
        def _():
            process(ix_hbm)

        @pl.when(c == 1)
        def _():
            process(iy_hbm)

    return run(ix, iy)


def _finalize_kernel(h_ref, st_ref, lam_ref, out_ref):
    xh = jnp.sum(h_ref[0], axis=-1)
    yh = jnp.sum(h_ref[1], axis=-1)
    st = st_ref[:, 0, :]

    xsum = st[:, 0:3]
    ysum = st[:, 5:8]
    xmean = xsum / _NPIX
    ymean = ysum / _NPIX
    xbal = xmean / (jnp.sum(xmean, axis=1, keepdims=True) + 1e-08)
    ybal = ymean / (jnp.sum(ymean, axis=1, keepdims=True) + 1e-08)
    cb = jnp.mean(jnp.abs(xbal - ybal))

    xhn = xh / jnp.sum(xh, axis=1, keepdims=True)
    yhn = yh / jnp.sum(yh, axis=1, keepdims=True)
    u = 1.0 / _BINS
    xvalid = st[:, 4:5] > st[:, 3:4]
    yvalid = st[:, 9:10] > st[:, 8:9]
    xhist = jnp.where(xvalid, xhn, u)
    yhist = jnp.where(yvalid, yhn, u)

    log_input = jnp.log(xhist + 1e-08)
    safe_t = jnp.where(yhist > 0, yhist, 1.0)
    kl_el = jnp.where(yhist > 0, yhist * (jnp.log(safe_t) - log_input), 0.0)
    kl = jnp.sum(kl_el) / 16.0

    out_ref[...] = (lam_ref[0, 0] * (cb + kl))[None, None]


def _finalize(hist, stats, lam):
    out = pl.pallas_call(
        _finalize_kernel,
        out_shape=jax.ShapeDtypeStruct((1, 1), jnp.float32),
    )(hist, stats, lam)
    return out[0, 0]


def kernel(x, y, lambda_cc):
    ix, iy, stats = _stage_a(x, y)
    ix = ix.reshape(x.shape[0], _NCHUNK, _CHUNK)
    iy = iy.reshape(x.shape[0], _NCHUNK, _CHUNK)
    hist = _sc_hist(ix, iy)
    return stats[0, 0, 0] * 0.0 + hist[0, 0, 0] * 0.0
    hist = hist.reshape(2, 16, _BINS, 16)
    lam = jnp.asarray(lambda_cc, jnp.float32).reshape(1, 1)
    return _finalize(hist, stats, lam)


# speedup vs baseline: 1.4179x; 1.4179x over previous
import functools

import jax
import jax.numpy as jnp
from jax import lax
from jax.experimental import pallas as pl
from jax.experimental.pallas import tpu as pltpu
from jax.experimental.pallas import tpu_sc as plsc

_BINS = 64
_H = 512
_W = 512
_NPIX = float(_H * _W)
_CHUNK = 16384
_NCHUNK = (_H * _W) // _CHUNK


def _dense_stats(img):
    r = img[0]
    g = img[1]
    b = img[2]
    s0 = jnp.sum(r)
    s1 = jnp.sum(g)
    s2 = jnp.sum(b)
    gray = 0.299 * r + 0.587 * g + 0.114 * b
    mn = jnp.min(gray)
    mx = jnp.max(gray)
    denom = mx - mn
    safe = jnp.where(denom > 0, denom, 1.0)
    xn = (gray - mn) / safe
    bidx = (xn * (_BINS - 1)).astype(jnp.int32)
    bidx = jnp.clip(bidx, 0, _BINS - 1)
    return s0, s1, s2, mn, mx, bidx * 16


def _stage_a_kernel(x_ref, y_ref, ix_ref, iy_ref, stats_ref):
    xs0, xs1, xs2, xmn, xmx, xidx = _dense_stats(x_ref[0])
    ys0, ys1, ys2, ymn, ymx, yidx = _dense_stats(y_ref[0])

    ix_ref[...] = xidx[None]
    iy_ref[...] = yidx[None]

    col = jax.lax.broadcasted_iota(jnp.int32, (1, 1, 16), 2)
    row = jnp.zeros((1, 1, 16), jnp.float32)
    for k, v in enumerate((xs0, xs1, xs2, xmn, xmx, ys0, ys1, ys2, ymn, ymx)):
        row = jnp.where(col == k, v, row)
    stats_ref[...] = row


def _stage_a(x, y):
    B = x.shape[0]
    return pl.pallas_call(
        _stage_a_kernel,
        grid=(B,),
        in_specs=[
            pl.BlockSpec((1, 3, _H, _W), lambda i: (i, 0, 0, 0)),
            pl.BlockSpec((1, 3, _H, _W), lambda i: (i, 0, 0, 0)),
        ],
        out_specs=[
            pl.BlockSpec((1, _H, _W), lambda i: (i, 0, 0)),
            pl.BlockSpec((1, _H, _W), lambda i: (i, 0, 0)),
            pl.BlockSpec((1, 1, 16), lambda i: (i, 0, 0)),
        ],
        out_shape=[
            jax.ShapeDtypeStruct((B, _H, _W), jnp.int32),
            jax.ShapeDtypeStruct((B, _H, _W), jnp.int32),
            jax.ShapeDtypeStruct((B, 1, 16), jnp.float32),
        ],
    )(x, y)


def _sc_hist(ix, iy):
    rows = _CHUNK // _W
    mesh = plsc.VectorSubcoreMesh(core_axis_name="c", subcore_axis_name="s")

    @functools.partial(
        pl.kernel,
        out_type=jax.ShapeDtypeStruct((2, 16, _BINS * 16), jnp.float32),
        mesh=mesh,
        scratch_types=[
            pltpu.VMEM((rows, _W), jnp.int32),
            pltpu.VMEM((rows, _W), jnp.int32),
            pltpu.VMEM((_BINS * 16,), jnp.float32),
            pltpu.SemaphoreType.DMA,
            pltpu.SemaphoreType.DMA,
        ],
        compiler_params=pltpu.CompilerParams(
            needs_layout_passes=False, use_tc_tiling_on_sc=True
        ),
    )
    def run(ix_hbm, iy_hbm, out_hbm, buf0_v, buf1_v, hist_v, sem0, sem1):
        c = lax.axis_index("c")
        s = lax.axis_index("s")
        zeros16 = jnp.zeros((16,), jnp.float32)
        ones16 = jnp.ones((16,), jnp.float32)
        lanes = lax.iota(jnp.int32, 16)

        def process(ghbm):
            for b in range(_BINS):
                hist_v[pl.ds(b * 16, 16)] = zeros16

            bufs = (buf0_v, buf1_v)
            sems = (sem0, sem1)
            handles = {
                0: pltpu.async_copy(ghbm.at[s, pl.ds(0, rows)], bufs[0], sems[0])
            }
            for ch in range(_NCHUNK):
                nxt = ch + 1
                if nxt < _NCHUNK:
                    handles[nxt] = pltpu.async_copy(
                        ghbm.at[s, pl.ds(nxt * rows, rows)],
                        bufs[nxt % 2],
                        sems[nxt % 2],
                    )
                handles[ch].wait()
                bufref = bufs[ch % 2]

                @plsc.parallel_loop(0, _CHUNK // 16, unroll=4)
                def _body(i, bufref=bufref):
                    r = i >> 5
                    col = (i & 31) << 4
                    bi16 = bufref[r, pl.ds(col, 16)]
                    plsc.addupdate_scatter(hist_v, [bi16 + lanes], ones16)

            pltpu.sync_copy(hist_v, out_hbm.at[c, s])

        @pl.when(c == 0)
        def _():
            process(ix_hbm)

        @pl.when(c == 1)
        def _():
            process(iy_hbm)

    return run(ix, iy)


def _finalize_kernel(h_ref, st_ref, lam_ref, out_ref):
    xh = jnp.sum(h_ref[0], axis=-1)
    yh = jnp.sum(h_ref[1], axis=-1)
    st = st_ref[:, 0, :]

    xsum = st[:, 0:3]
    ysum = st[:, 5:8]
    xmean = xsum / _NPIX
    ymean = ysum / _NPIX
    xbal = xmean / (jnp.sum(xmean, axis=1, keepdims=True) + 1e-08)
    ybal = ymean / (jnp.sum(ymean, axis=1, keepdims=True) + 1e-08)
    cb = jnp.mean(jnp.abs(xbal - ybal))

    xhn = xh / jnp.sum(xh, axis=1, keepdims=True)
    yhn = yh / jnp.sum(yh, axis=1, keepdims=True)
    u = 1.0 / _BINS
    xvalid = st[:, 4:5] > st[:, 3:4]
    yvalid = st[:, 9:10] > st[:, 8:9]
    xhist = jnp.where(xvalid, xhn, u)
    yhist = jnp.where(yvalid, yhn, u)

    log_input = jnp.log(xhist + 1e-08)
    safe_t = jnp.where(yhist > 0, yhist, 1.0)
    kl_el = jnp.where(yhist > 0, yhist * (jnp.log(safe_t) - log_input), 0.0)
    kl = jnp.sum(kl_el) / 16.0

    out_ref[...] = (lam_ref[0, 0] * (cb + kl))[None, None]


def _finalize(hist, stats, lam):
    out = pl.pallas_call(
        _finalize_kernel,
        out_shape=jax.ShapeDtypeStruct((1, 1), jnp.float32),
    )(hist, stats, lam)
    return out[0, 0]


def kernel(x, y, lambda_cc):
    ix, iy, stats = _stage_a(x, y)
    hist = _sc_hist(ix, iy)
    hist = hist.reshape(2, 16, _BINS, 16)
    lam = jnp.asarray(lambda_cc, jnp.float32).reshape(1, 1)
    return _finalize(hist, stats, lam)
    hist = hist.reshape(2, 16, _BINS, 16)
    lam = jnp.asarray(lambda_cc, jnp.float32).reshape(1, 1)
    return _finalize(hist, stats, lam)


# speedup vs baseline: 1.4937x; 1.0535x over previous
import functools

import jax
import jax.numpy as jnp
from jax import lax
from jax.experimental import pallas as pl
from jax.experimental.pallas import tpu as pltpu
from jax.experimental.pallas import tpu_sc as plsc

_BINS = 64
_H = 512
_W = 512
_NPIX = float(_H * _W)
_CHUNK = 16384
_NCHUNK = (_H * _W) // _CHUNK
_ROWS = _CHUNK // _W
_HB = 8


def _dense_stats(img):
    r = img[0]
    g = img[1]
    b = img[2]
    s0 = jnp.sum(r)
    s1 = jnp.sum(g)
    s2 = jnp.sum(b)
    gray = 0.299 * r + 0.587 * g + 0.114 * b
    mn = jnp.min(gray)
    mx = jnp.max(gray)
    denom = mx - mn
    safe = jnp.where(denom > 0, denom, 1.0)
    xn = (gray - mn) / safe
    bidx = (xn * (_BINS - 1)).astype(jnp.int32)
    bidx = jnp.clip(bidx, 0, _BINS - 1)
    return s0, s1, s2, mn, mx, bidx * 16


def _stage_a_kernel(x_ref, y_ref, ix_ref, iy_ref, stats_ref):
    xs0, xs1, xs2, xmn, xmx, xidx = _dense_stats(x_ref[0])
    ys0, ys1, ys2, ymn, ymx, yidx = _dense_stats(y_ref[0])

    ix_ref[...] = xidx[None]
    iy_ref[...] = yidx[None]

    col = jax.lax.broadcasted_iota(jnp.int32, (1, 1, 16), 2)
    row = jnp.zeros((1, 1, 16), jnp.float32)
    for k, v in enumerate((xs0, xs1, xs2, xmn, xmx, ys0, ys1, ys2, ymn, ymx)):
        row = jnp.where(col == k, v, row)
    stats_ref[...] = row


def _stage_a(x, y, off):
    return pl.pallas_call(
        _stage_a_kernel,
        grid=(_HB,),
        in_specs=[
            pl.BlockSpec((1, 3, _H, _W), lambda i: (i + off, 0, 0, 0)),
            pl.BlockSpec((1, 3, _H, _W), lambda i: (i + off, 0, 0, 0)),
        ],
        out_specs=[
            pl.BlockSpec((1, _H, _W), lambda i: (i, 0, 0)),
            pl.BlockSpec((1, _H, _W), lambda i: (i, 0, 0)),
            pl.BlockSpec((1, 1, 16), lambda i: (i, 0, 0)),
        ],
        out_shape=[
            jax.ShapeDtypeStruct((_HB, _H, _W), jnp.int32),
            jax.ShapeDtypeStruct((_HB, _H, _W), jnp.int32),
            jax.ShapeDtypeStruct((_HB, 1, 16), jnp.float32),
        ],
    )(x, y)


def _sc_hist(ix, iy):
    nch = _NCHUNK // 2
    mesh = plsc.VectorSubcoreMesh(core_axis_name="c", subcore_axis_name="s")

    @functools.partial(
        pl.kernel,
        out_type=jax.ShapeDtypeStruct((2, 16, _BINS * 16), jnp.float32),
        mesh=mesh,
        scratch_types=[
            pltpu.VMEM((_ROWS, _W), jnp.int32),
            pltpu.VMEM((_ROWS, _W), jnp.int32),
            pltpu.VMEM((_BINS * 16,), jnp.float32),
            pltpu.SemaphoreType.DMA,
            pltpu.SemaphoreType.DMA,
        ],
        compiler_params=pltpu.CompilerParams(
            needs_layout_passes=False, use_tc_tiling_on_sc=True
        ),
    )
    def run(ix_hbm, iy_hbm, out_hbm, buf0_v, buf1_v, hist_v, sem0, sem1):
        c = lax.axis_index("c")
        s = lax.axis_index("s")
        img = s >> 1
        base_row = (s & 1) * (nch * _ROWS)
        zeros16 = jnp.zeros((16,), jnp.float32)
        ones16 = jnp.ones((16,), jnp.float32)
        lanes = lax.iota(jnp.int32, 16)

        def process(ghbm):
            for b in range(_BINS):
                hist_v[pl.ds(b * 16, 16)] = zeros16

            bufs = (buf0_v, buf1_v)
            sems = (sem0, sem1)
            handles = {
                0: pltpu.async_copy(
                    ghbm.at[img, pl.ds(base_row, _ROWS)], bufs[0], sems[0]
                )
            }
            for ch in range(nch):
                nxt = ch + 1
                if nxt < nch:
                    handles[nxt] = pltpu.async_copy(
                        ghbm.at[img, pl.ds(base_row + nxt * _ROWS, _ROWS)],
                        bufs[nxt % 2],
                        sems[nxt % 2],
                    )
                handles[ch].wait()
                bufref = bufs[ch % 2]

                @plsc.parallel_loop(0, _CHUNK // 16, unroll=4)
                def _body(i, bufref=bufref):
                    r = i >> 5
                    col = (i & 31) << 4
                    bi16 = bufref[r, pl.ds(col, 16)]
                    plsc.addupdate_scatter(hist_v, [bi16 + lanes], ones16)

            pltpu.sync_copy(hist_v, out_hbm.at[c, s])

        @pl.when(c == 0)
        def _():
            process(ix_hbm)

        @pl.when(c == 1)
        def _():
            process(iy_hbm)

    return run(ix, iy)


def _half_hists(h_ref, t):
    g = jnp.sum(h_ref[t], axis=-1)
    return g[:, 0:_BINS] + g[:, _BINS : 2 * _BINS]


def _finalize_kernel(h1_ref, h2_ref, st1_ref, st2_ref, lam_ref, out_ref):
    xh = jnp.concatenate([_half_hists(h1_ref, 0), _half_hists(h2_ref, 0)], axis=0)
    yh = jnp.concatenate([_half_hists(h1_ref, 1), _half_hists(h2_ref, 1)], axis=0)
    st = jnp.concatenate([st1_ref[:, 0, :], st2_ref[:, 0, :]], axis=0)

    xsum = st[:, 0:3]
    ysum = st[:, 5:8]
    xmean = xsum / _NPIX
    ymean = ysum / _NPIX
    xbal = xmean / (jnp.sum(xmean, axis=1, keepdims=True) + 1e-08)
    ybal = ymean / (jnp.sum(ymean, axis=1, keepdims=True) + 1e-08)
    cb = jnp.mean(jnp.abs(xbal - ybal))

    xhn = xh / jnp.sum(xh, axis=1, keepdims=True)
    yhn = yh / jnp.sum(yh, axis=1, keepdims=True)
    u = 1.0 / _BINS
    xvalid = st[:, 4:5] > st[:, 3:4]
    yvalid = st[:, 9:10] > st[:, 8:9]
    xhist = jnp.where(xvalid, xhn, u)
    yhist = jnp.where(yvalid, yhn, u)

    log_input = jnp.log(xhist + 1e-08)
    safe_t = jnp.where(yhist > 0, yhist, 1.0)
    kl_el = jnp.where(yhist > 0, yhist * (jnp.log(safe_t) - log_input), 0.0)
    kl = jnp.sum(kl_el) / 16.0

    out_ref[...] = (lam_ref[0, 0] * (cb + kl))[None, None]


def _finalize(h1, h2, st1, st2, lam):
    out = pl.pallas_call(
        _finalize_kernel,
        out_shape=jax.ShapeDtypeStruct((1, 1), jnp.float32),
    )(h1, h2, st1, st2, lam)
    return out[0, 0]


def kernel(x, y, lambda_cc):
    ix1, iy1, st1 = _stage_a(x, y, 0)
    hist1 = _sc_hist(ix1, iy1)
    ix2, iy2, st2 = _stage_a(x, y, _HB)
    hist2 = _sc_hist(ix2, iy2)
    h1 = hist1.reshape(2, _HB, 2 * _BINS, 16)
    h2 = hist2.reshape(2, _HB, 2 * _BINS, 16)
    lam = jnp.asarray(lambda_cc, jnp.float32).reshape(1, 1)
    return _finalize(h1, h2, st1, st2, lam)


# speedup vs baseline: 1.7210x; 1.1522x over previous
import functools

import jax
import jax.numpy as jnp
from jax import lax
from jax.experimental import pallas as pl
from jax.experimental.pallas import tpu as pltpu
from jax.experimental.pallas import tpu_sc as plsc

_BINS = 64
_H = 512
_W = 512
_NPIX = float(_H * _W)
_CHUNK = 16384
_NCHUNK = (_H * _W) // _CHUNK
_ROWS = _CHUNK // _W
_HB = 8


def _dense_stats(img):
    r = img[0]
    g = img[1]
    b = img[2]
    s0 = jnp.sum(r)
    s1 = jnp.sum(g)
    s2 = jnp.sum(b)
    gray = 0.299 * r + 0.587 * g + 0.114 * b
    mn = jnp.min(gray)
    mx = jnp.max(gray)
    denom = mx - mn
    safe = jnp.where(denom > 0, denom, 1.0)
    xn = (gray - mn) / safe
    bidx = (xn * (_BINS - 1)).astype(jnp.int32)
    bidx = jnp.clip(bidx, 0, _BINS - 1)
    q = (
        bidx[:, 0:128]
        | (bidx[:, 128:256] << 8)
        | (bidx[:, 256:384] << 16)
        | (bidx[:, 384:512] << 24)
    )
    return s0, s1, s2, mn, mx, q


def _stage_a_kernel(x_ref, y_ref, ix_ref, iy_ref, stats_ref):
    xs0, xs1, xs2, xmn, xmx, xidx = _dense_stats(x_ref[0])
    ys0, ys1, ys2, ymn, ymx, yidx = _dense_stats(y_ref[0])

    ix_ref[...] = xidx[None]
    iy_ref[...] = yidx[None]

    col = jax.lax.broadcasted_iota(jnp.int32, (1, 1, 16), 2)
    row = jnp.zeros((1, 1, 16), jnp.float32)
    for k, v in enumerate((xs0, xs1, xs2, xmn, xmx, ys0, ys1, ys2, ymn, ymx)):
        row = jnp.where(col == k, v, row)
    stats_ref[...] = row


def _stage_a(x, y, off):
    return pl.pallas_call(
        _stage_a_kernel,
        grid=(_HB,),
        in_specs=[
            pl.BlockSpec((1, 3, _H, _W), lambda i: (i + off, 0, 0, 0)),
            pl.BlockSpec((1, 3, _H, _W), lambda i: (i + off, 0, 0, 0)),
        ],
        out_specs=[
            pl.BlockSpec((1, _H, _W // 4), lambda i: (i, 0, 0)),
            pl.BlockSpec((1, _H, _W // 4), lambda i: (i, 0, 0)),
            pl.BlockSpec((1, 1, 16), lambda i: (i, 0, 0)),
        ],
        out_shape=[
            jax.ShapeDtypeStruct((_HB, _H, _W // 4), jnp.int32),
            jax.ShapeDtypeStruct((_HB, _H, _W // 4), jnp.int32),
            jax.ShapeDtypeStruct((_HB, 1, 16), jnp.float32),
        ],
    )(x, y)


def _sc_hist(ix, iy):
    rows = 32
    nch = (_H // 2) // rows
    mesh = plsc.VectorSubcoreMesh(core_axis_name="c", subcore_axis_name="s")

    @functools.partial(
        pl.kernel,
        out_type=jax.ShapeDtypeStruct((2, 16, _BINS * 16), jnp.float32),
        mesh=mesh,
        scratch_types=[
            pltpu.VMEM((rows, _W // 4), jnp.int32),
            pltpu.VMEM((rows, _W // 4), jnp.int32),
            pltpu.VMEM((_BINS * 16,), jnp.float32),
            pltpu.SemaphoreType.DMA,
            pltpu.SemaphoreType.DMA,
        ],
        compiler_params=pltpu.CompilerParams(
            needs_layout_passes=False, use_tc_tiling_on_sc=True
        ),
    )
    def run(ix_hbm, iy_hbm, out_hbm, buf0_v, buf1_v, hist_v, sem0, sem1):
        c = lax.axis_index("c")
        s = lax.axis_index("s")
        img = s >> 1
        base_row = (s & 1) * (nch * rows)
        zeros16 = jnp.zeros((16,), jnp.float32)
        ones16 = jnp.ones((16,), jnp.float32)
        lanes = lax.iota(jnp.int32, 16)
        m = jnp.full((16,), 0x3F0, jnp.int32)

        def process(ghbm):
            for b in range(_BINS):
                hist_v[pl.ds(b * 16, 16)] = zeros16

            bufs = (buf0_v, buf1_v)
            sems = (sem0, sem1)
            handles = {
                0: pltpu.async_copy(
                    ghbm.at[img, pl.ds(base_row, rows)], bufs[0], sems[0]
                )
            }
            for ch in range(nch):
                nxt = ch + 1
                if nxt < nch:
                    handles[nxt] = pltpu.async_copy(
                        ghbm.at[img, pl.ds(base_row + nxt * rows, rows)],
                        bufs[nxt % 2],
                        sems[nxt % 2],
                    )
                handles[ch].wait()
                bufref = bufs[ch % 2]

                @plsc.parallel_loop(0, rows * (_W // 4) // 16, unroll=4)
                def _body(i, bufref=bufref):
                    r = i >> 3
                    col = (i & 7) << 4
                    v = bufref[r, pl.ds(col, 16)]
                    plsc.addupdate_scatter(hist_v, [((v << 4) & m) + lanes], ones16)
                    plsc.addupdate_scatter(hist_v, [((v >> 4) & m) + lanes], ones16)
                    plsc.addupdate_scatter(hist_v, [((v >> 12) & m) + lanes], ones16)
                    plsc.addupdate_scatter(hist_v, [((v >> 20) & m) + lanes], ones16)

            pltpu.sync_copy(hist_v, out_hbm.at[c, s])

        @pl.when(c == 0)
        def _():
            process(ix_hbm)

        @pl.when(c == 1)
        def _():
            process(iy_hbm)

    return run(ix, iy)


def _half_hists(h_ref, t):
    g = jnp.sum(h_ref[t], axis=-1)
    return g[:, 0:_BINS] + g[:, _BINS : 2 * _BINS]


def _finalize_kernel(h1_ref, h2_ref, st1_ref, st2_ref, lam_ref, out_ref):
    xh = jnp.concatenate([_half_hists(h1_ref, 0), _half_hists(h2_ref, 0)], axis=0)
    yh = jnp.concatenate([_half_hists(h1_ref, 1), _half_hists(h2_ref, 1)], axis=0)
    st = jnp.concatenate([st1_ref[:, 0, :], st2_ref[:, 0, :]], axis=0)

    xsum = st[:, 0:3]
    ysum = st[:, 5:8]
    xmean = xsum / _NPIX
    ymean = ysum / _NPIX
    xbal = xmean / (jnp.sum(xmean, axis=1, keepdims=True) + 1e-08)
    ybal = ymean / (jnp.sum(ymean, axis=1, keepdims=True) + 1e-08)
    cb = jnp.mean(jnp.abs(xbal - ybal))

    xhn = xh / jnp.sum(xh, axis=1, keepdims=True)
    yhn = yh / jnp.sum(yh, axis=1, keepdims=True)
    u = 1.0 / _BINS
    xvalid = st[:, 4:5] > st[:, 3:4]
    yvalid = st[:, 9:10] > st[:, 8:9]
    xhist = jnp.where(xvalid, xhn, u)
    yhist = jnp.where(yvalid, yhn, u)

    log_input = jnp.log(xhist + 1e-08)
    safe_t = jnp.where(yhist > 0, yhist, 1.0)
    kl_el = jnp.where(yhist > 0, yhist * (jnp.log(safe_t) - log_input), 0.0)
    kl = jnp.sum(kl_el) / 16.0

    out_ref[...] = (lam_ref[0, 0] * (cb + kl))[None, None]


def _finalize(h1, h2, st1, st2, lam):
    out = pl.pallas_call(
        _finalize_kernel,
        out_shape=jax.ShapeDtypeStruct((1, 1), jnp.float32),
    )(h1, h2, st1, st2, lam)
    return out[0, 0]


def kernel(x, y, lambda_cc):
    ix1, iy1, st1 = _stage_a(x, y, 0)
    hist1 = _sc_hist(ix1, iy1)
    ix2, iy2, st2 = _stage_a(x, y, _HB)
    hist2 = _sc_hist(ix2, iy2)
    h1 = hist1.reshape(2, _HB, 2 * _BINS, 16)
    h2 = hist2.reshape(2, _HB, 2 * _BINS, 16)
    lam = jnp.asarray(lambda_cc, jnp.float32).reshape(1, 1)
    return _finalize(h1, h2, st1, st2, lam)
